# baseline (speedup 1.0000x reference)
import functools

import jax
import jax.numpy as jnp
from jax import lax
from jax.experimental import pallas as pl
from jax.experimental.pallas import tpu as pltpu
from jax.experimental.pallas import tpu_sc as plsc

_NC, _NS, _L = 2, 16, 16
_NW = _NC * _NS
_C = 16
_U = 8


def _make_sc_kernel(bs, sl, d):
    spw = sl // _NW
    nchunk = spw // _C
    nsteps = nchunk * bs
    mesh = plsc.VectorSubcoreMesh(core_axis_name="c", subcore_axis_name="s")

    @functools.partial(
        pl.kernel,
        out_type=jax.ShapeDtypeStruct((bs * sl, d), jnp.float32),
        mesh=mesh,
        scratch_types=[
            pltpu.VMEM((_C, d), jnp.float32),
            pltpu.VMEM((_C, d), jnp.float32),
            pltpu.VMEM((_C, d), jnp.float32),
            pltpu.VMEM((bs * spw,), jnp.float32),
            pltpu.SemaphoreType.DMA,
            pltpu.SemaphoreType.DMA,
            pltpu.SemaphoreType.DMA,
            pltpu.SemaphoreType.DMA,
        ],
    )
    def sc_pe(x_hbm, mask_hbm, pe_hbm, out_hbm,
              pe_v, x0, x1, m_v, sx0, sx1, ss0, ss1):
        wid = lax.axis_index("s") * _NC + lax.axis_index("c")
        s0 = wid * spw
        for b in range(bs):
            pltpu.sync_copy(
                mask_hbm.at[pl.ds(b * sl + s0, spw)],
                m_v.at[pl.ds(b * spw, spw)],
            )

        xb = (x0, x1)
        sx = (sx0, sx1)
        ss = (ss0, ss1)

        def row_start(t):
            return (t % bs) * sl + s0 + (t // bs) * _C

        pltpu.async_copy(x_hbm.at[pl.ds(row_start(0), _C)], x0, sx0)

        def wait_dma(buf, sem):
            pltpu.make_async_copy(x_hbm.at[pl.ds(0, _C)], buf, sem).wait()

        def step_body(i, carry):
            for p in (0, 1):
                t = 2 * i + p
                ci = t // bs
                b = t % bs

                @pl.when(b == 0)
                def _load_pe():
                    pltpu.sync_copy(pe_hbm.at[pl.ds(s0 + ci * _C, _C)], pe_v)

                tn = t + 1

                @pl.when(tn < nsteps)
                def _prefetch():
                    @pl.when(t >= 1)
                    def _wait_prev_store():
                        wait_dma(xb[1 - p], ss[1 - p])

                    pltpu.async_copy(
                        x_hbm.at[pl.ds(row_start(tn), _C)], xb[1 - p], sx[1 - p]
                    )

                wait_dma(xb[p], sx[p])

                m_vec = m_v[pl.ds(b * spw + ci * _C, _L)]
                xv = xb[p]
                for j in range(_C):
                    m16 = m_vec.at[jnp.full((_L,), j, jnp.int32)].get(
                        mode="promise_in_bounds"
                    )

                    def col_body(kk, cc, j=j, xv=xv, m16=m16):
                        base = kk * (_L * _U)
                        for u in range(_U):
                            s16 = pl.ds(base + u * _L, _L)
                            xv[j, s16] = (xv[j, s16] + pe_v[j, s16]) * m16
                        return cc

                    lax.fori_loop(0, d // (_L * _U), col_body, 0)

                pltpu.async_copy(xv, out_hbm.at[pl.ds(row_start(t), _C)], ss[p])
            return carry

        lax.fori_loop(0, nsteps // 2, step_body, 0)
        wait_dma(x0, ss0)
        wait_dma(x1, ss1)

    return sc_pe


def kernel(x, mask, pos_emb):
    bs, sl, d = x.shape
    out = _make_sc_kernel(bs, sl, d)(
        x.reshape(bs * sl, d), mask.reshape(bs * sl), pos_emb
    )
    return out.reshape(bs, sl, d)


# speedup vs baseline: 1.1422x; 1.1422x over previous
import functools

import jax
import jax.numpy as jnp
from jax import lax
from jax.experimental import pallas as pl
from jax.experimental.pallas import tpu as pltpu
from jax.experimental.pallas import tpu_sc as plsc

S_BLK = 1024
S_TC = 3072

_NC, _NS, _L = 2, 16, 16
_NW = _NC * _NS
_C = 16
_U = 8


def _tc_kernel_body(x_ref, mask_ref, pe_ref, out_ref):
    m = mask_ref[0, 0, 0, :]
    out_ref[...] = (x_ref[...] + pe_ref[...]) * m[:, None]


def _make_sc_kernel(bs, sl, d, s_lo):
    span = sl - s_lo
    spw = span // _NW
    nchunk = spw // _C
    nsteps = nchunk * bs
    mesh = plsc.VectorSubcoreMesh(core_axis_name="c", subcore_axis_name="s")

    @functools.partial(
        pl.kernel,
        out_type=jax.ShapeDtypeStruct((bs * span, d), jnp.float32),
        mesh=mesh,
        scratch_types=[
            pltpu.VMEM((_C, d), jnp.float32),
            pltpu.VMEM((_C, d), jnp.float32),
            pltpu.VMEM((_C, d), jnp.float32),
            pltpu.VMEM((bs * spw,), jnp.float32),
            pltpu.SemaphoreType.DMA,
            pltpu.SemaphoreType.DMA,
            pltpu.SemaphoreType.DMA,
            pltpu.SemaphoreType.DMA,
        ],
    )
    def sc_pe(x_hbm, mask_hbm, pe_hbm, out_hbm,
              pe_v, x0, x1, m_v, sx0, sx1, ss0, ss1):
        wid = lax.axis_index("s") * _NC + lax.axis_index("c")
        s0 = s_lo + wid * spw
        o0 = wid * spw
        for b in range(bs):
            pltpu.sync_copy(
                mask_hbm.at[pl.ds(b * sl + s0, spw)],
                m_v.at[pl.ds(b * spw, spw)],
            )

        xb = (x0, x1)
        sx = (sx0, sx1)
        ss = (ss0, ss1)

        def x_start(t):
            return (t % bs) * sl + s0 + (t // bs) * _C

        def o_start(t):
            return (t % bs) * span + o0 + (t // bs) * _C

        pltpu.async_copy(x_hbm.at[pl.ds(x_start(0), _C)], x0, sx0)

        def wait_dma(buf, sem):
            pltpu.make_async_copy(x_hbm.at[pl.ds(0, _C)], buf, sem).wait()

        def step_body(i, carry):
            for p in (0, 1):
                t = 2 * i + p
                ci = t // bs
                b = t % bs

                @pl.when(b == 0)
                def _load_pe():
                    pltpu.sync_copy(pe_hbm.at[pl.ds(s0 + ci * _C, _C)], pe_v)

                tn = t + 1

                @pl.when(tn < nsteps)
                def _prefetch():
                    @pl.when(t >= 1)
                    def _wait_prev_store():
                        wait_dma(xb[1 - p], ss[1 - p])

                    pltpu.async_copy(
                        x_hbm.at[pl.ds(x_start(tn), _C)], xb[1 - p], sx[1 - p]
                    )

                wait_dma(xb[p], sx[p])

                m_vec = m_v[pl.ds(b * spw + ci * _C, _L)]
                xv = xb[p]
                for j in range(_C):
                    m16 = m_vec.at[jnp.full((_L,), j, jnp.int32)].get(
                        mode="promise_in_bounds"
                    )

                    def col_body(kk, cc, j=j, xv=xv, m16=m16):
                        base = kk * (_L * _U)
                        for u in range(_U):
                            s16 = pl.ds(base + u * _L, _L)
                            xv[j, s16] = (xv[j, s16] + pe_v[j, s16]) * m16
                        return cc

                    lax.fori_loop(0, d // (_L * _U), col_body, 0)

                pltpu.async_copy(xv, out_hbm.at[pl.ds(o_start(t), _C)], ss[p])
            return carry

        lax.fori_loop(0, nsteps // 2, step_body, 0)
        wait_dma(x0, ss0)
        wait_dma(x1, ss1)

    return sc_pe


def kernel(x, mask, pos_emb):
    bs, sl, d = x.shape
    span = sl - S_TC

    mask4 = mask.reshape(bs, sl // S_BLK, 1, S_BLK)
    out_tc = pl.pallas_call(
        _tc_kernel_body,
        grid=(S_TC // S_BLK, bs),
        in_specs=[
            pl.BlockSpec((1, S_BLK, d), lambda s, b: (b, s, 0)),
            pl.BlockSpec((1, 1, 1, S_BLK), lambda s, b: (b, s, 0, 0)),
            pl.BlockSpec((S_BLK, d), lambda s, b: (s, 0)),
        ],
        out_specs=pl.BlockSpec((1, S_BLK, d), lambda s, b: (b, s, 0)),
        out_shape=jax.ShapeDtypeStruct((bs, sl, d), x.dtype),
    )(x, mask4, pos_emb)

    sc_piece = _make_sc_kernel(bs, sl, d, S_TC)(
        x.reshape(bs * sl, d), mask.reshape(bs * sl), pos_emb
    ).reshape(bs, span, d)

    return lax.dynamic_update_slice(out_tc, sc_piece, (0, S_TC, 0))


# speedup vs baseline: 1.6403x; 1.4360x over previous
import jax
import jax.numpy as jnp
from jax.experimental import pallas as pl

S_BLK = 512
B_BLK = 2


def _pe_kernel(x_ref, mask_ref, pe_ref, out_ref):
    m = mask_ref[:, 0, 0, :]
    out_ref[...] = (x_ref[...] + pe_ref[...]) * m[:, :, None]


def kernel(x, mask, pos_emb):
    bs, sl, d = x.shape
    grid = (sl // S_BLK, bs // B_BLK)
    mask4 = mask.reshape(bs, sl // S_BLK, 1, S_BLK)
    return pl.pallas_call(
        _pe_kernel,
        grid=grid,
        in_specs=[
            pl.BlockSpec((B_BLK, S_BLK, d), lambda s, b: (b, s, 0)),
            pl.BlockSpec((B_BLK, 1, 1, S_BLK), lambda s, b: (b, s, 0, 0)),
            pl.BlockSpec((S_BLK, d), lambda s, b: (s, 0)),
        ],
        out_specs=pl.BlockSpec((B_BLK, S_BLK, d), lambda s, b: (b, s, 0)),
        out_shape=jax.ShapeDtypeStruct((bs, sl, d), x.dtype),
    )(x, mask4, pos_emb)
